# baseline (speedup 1.0000x reference)
import functools

import jax
import jax.numpy as jnp
from jax import lax
from jax.experimental import pallas as pl
from jax.experimental.pallas import tpu as pltpu
from jax.experimental.pallas import tpu_sc as plsc

_MARGIN_M = 1.0
_MARGIN_T = 1.0

_SC_NC = 2
_SC_NS = 16
_SC_NW = _SC_NC * _SC_NS
_SC_CHUNK = 50000
_SC_UNROLL = 25
_SC_ACCS = 5


def _sc_chunk_fixup(buf, lab, chunk_base):
    in_c = jnp.logical_and(lab >= chunk_base, lab < chunk_base + _SC_CHUNK)
    off = jnp.where(in_c, lab - chunk_base, 0)
    sbase = (off // 16) * 16
    lane = off - sbase
    sl = buf[pl.ds(sbase, 16)]
    mask = jnp.logical_and(lax.iota(jnp.int32, 16) == lane, in_c)
    fy = jnp.max(jnp.where(mask, sl, -3.4e38))
    fy = jnp.where(in_c, fy, 0.0)
    buf[pl.ds(sbase, 16)] = jnp.where(mask, -1e10, sl)
    return fy


def _sc_chunk_max(buf):
    span = 16 * _SC_UNROLL
    n_steps = _SC_CHUNK // span

    def step(i, accs):
        base = i * span
        out = list(accs)
        for j in range(_SC_UNROLL):
            v = buf[pl.ds(base + j * 16, 16)]
            out[j % _SC_ACCS] = jnp.maximum(out[j % _SC_ACCS], v)
        return tuple(out)

    init = tuple(jnp.full((16,), -3.4e38, jnp.float32)
                 for _ in range(_SC_ACCS))
    accs = lax.fori_loop(0, n_steps, step, init)
    m = accs[0]
    for j in range(1, _SC_ACCS):
        m = jnp.maximum(m, accs[j])
    return jnp.max(m)


def _sc_worker(rpw, ncls, label_hbm, pred_hbm, out_hbm,
               labels_v, buf0, buf1, res_v, sem0, sem1):
    cid = lax.axis_index("c")
    sid = lax.axis_index("s")
    wid = sid * _SC_NC + cid
    base_row = wid * rpw

    pltpu.sync_copy(label_hbm.at[pl.ds(base_row, rpw)], labels_v)
    pltpu.make_async_copy(
        pred_hbm.at[pl.ds(base_row * ncls, _SC_CHUNK)], buf0, sem0).start()

    def row_body(p, wsum):
        r = base_row + p
        lslice = labels_v[pl.ds((p // 16) * 16, 16)]
        lab = jnp.max(jnp.where(lax.iota(jnp.int32, 16) == p % 16,
                                lslice.astype(jnp.float32), -1.0)
                      ).astype(jnp.int32)
        row0 = r * ncls
        pltpu.make_async_copy(
            pred_hbm.at[pl.ds(row0 + _SC_CHUNK, _SC_CHUNK)],
            buf1, sem1).start()
        pltpu.make_async_copy(
            pred_hbm.at[pl.ds(row0, _SC_CHUNK)], buf0, sem0).wait()
        fy0 = _sc_chunk_fixup(buf0, lab, 0)
        m0 = _sc_chunk_max(buf0)

        @pl.when(p < rpw - 1)
        def _next():
            pltpu.make_async_copy(
                pred_hbm.at[pl.ds(row0 + ncls, _SC_CHUNK)],
                buf0, sem0).start()

        pltpu.make_async_copy(
            pred_hbm.at[pl.ds(row0 + _SC_CHUNK, _SC_CHUNK)],
            buf1, sem1).wait()
        fy1 = _sc_chunk_fixup(buf1, lab, _SC_CHUNK)
        m1 = _sc_chunk_max(buf1)

        fy = fy0 + fy1
        fnym = jnp.maximum(m0, m1)
        l = (jnp.maximum(_MARGIN_M + _MARGIN_T - fy, 0.0)
             + jnp.maximum(_MARGIN_M + fnym, 0.0))
        return wsum + l

    wsum = lax.fori_loop(0, rpw, row_body, 0.0)
    res_v[...] = jnp.where(lax.iota(jnp.int32, 16) == 0, wsum, 0.0)
    pltpu.sync_copy(res_v, out_hbm.at[pl.ds(wid * 16, 16)])


def kernel(prediction, label):
    nrows, ncls = prediction.shape
    rpw = nrows // _SC_NW

    mesh = plsc.VectorSubcoreMesh(core_axis_name="c", subcore_axis_name="s")
    sc_loss = functools.partial(
        pl.kernel,
        out_type=jax.ShapeDtypeStruct((_SC_NW * 16,), jnp.float32),
        mesh=mesh,
        scratch_types=[
            pltpu.VMEM((rpw,), jnp.int32),
            pltpu.VMEM((_SC_CHUNK,), jnp.float32),
            pltpu.VMEM((_SC_CHUNK,), jnp.float32),
            pltpu.VMEM((16,), jnp.float32),
            pltpu.SemaphoreType.DMA,
            pltpu.SemaphoreType.DMA,
        ],
        compiler_params=pltpu.CompilerParams(needs_layout_passes=False),
    )(functools.partial(_sc_worker, rpw, ncls))

    partial_sums = sc_loss(label, prediction.reshape(-1))
    return jnp.sum(partial_sums) / nrows
